# baseline (speedup 1.0000x reference)
import functools

import jax
import jax.numpy as jnp
from jax.experimental import pallas as pl
from jax.experimental.pallas import tpu as pltpu

_K_TOP = 16
_NEG = -3.0e38


def _mask_kernel(q_ref, k_ref, out_ref, *, k_top):
    q = q_ref[0]
    k = k_ref[0]
    qn = q / jnp.maximum(jnp.sqrt(jnp.sum(q * q, axis=-1, keepdims=True)), 1e-12)
    kn = k / jnp.maximum(jnp.sqrt(jnp.sum(k * k, axis=-1, keepdims=True)), 1e-12)
    scores = jax.lax.dot_general(
        qn, kn, (((1,), (1,)), ((), ())),
        preferred_element_type=jnp.float32,
        precision=jax.lax.Precision.DEFAULT,
    )

    m = jnp.max(scores, axis=-1, keepdims=True)
    for _ in range(k_top - 1):
        m = jnp.max(jnp.where(scores < m, scores, _NEG), axis=-1, keepdims=True)
    out_ref[0] = (scores >= m).astype(jnp.float32)


def kernel(q_blocks, k_blocks):
    B, Qb, C = q_blocks.shape
    _, Bb, _ = k_blocks.shape
    k_top = min(_K_TOP, Bb)
    tq = min(512, Qb)
    grid = (B, Qb // tq)
    return pl.pallas_call(
        functools.partial(_mask_kernel, k_top=k_top),
        grid=grid,
        in_specs=[
            pl.BlockSpec((1, tq, C), lambda b, qt: (b, qt, 0)),
            pl.BlockSpec((1, Bb, C), lambda b, qt: (b, 0, 0)),
        ],
        out_specs=pl.BlockSpec((1, tq, Bb), lambda b, qt: (b, qt, 0)),
        out_shape=jax.ShapeDtypeStruct((B, Qb, Bb), jnp.float32),
        compiler_params=pltpu.CompilerParams(
            dimension_semantics=("parallel", "parallel"),
        ),
    )(q_blocks, k_blocks)


# speedup vs baseline: 1.2001x; 1.2001x over previous
import functools

import jax
import jax.numpy as jnp
from jax.experimental import pallas as pl
from jax.experimental.pallas import tpu as pltpu

_K_TOP = 16
_NEG = -3.0e38


def _rank4(x):
    h = x.shape[-1] // 2
    a, b = x[..., :h], x[..., h:]
    s1, s2 = jnp.maximum(a, b), jnp.minimum(a, b)
    h //= 2
    a1, b1 = s1[..., :h], s1[..., h:]
    a2, b2 = s2[..., :h], s2[..., h:]
    t = jnp.minimum(a1, b1)
    u = jnp.maximum(a2, b2)
    v = (jnp.maximum(a1, b1), jnp.maximum(t, u),
         jnp.minimum(t, u), jnp.minimum(a2, b2))
    while h > 128:
        h //= 2
        a = [c[..., :h] for c in v]
        b = [c[..., h:] for c in v]
        z1, z2 = jnp.maximum(a[0], b[3]), jnp.maximum(a[1], b[2])
        z3, z4 = jnp.maximum(a[2], b[1]), jnp.maximum(a[3], b[0])
        p1, p2 = jnp.maximum(z1, z3), jnp.maximum(z2, z4)
        q1, q2 = jnp.minimum(z1, z3), jnp.minimum(z2, z4)
        v = (jnp.maximum(p1, p2), jnp.minimum(p1, p2),
             jnp.maximum(q1, q2), jnp.minimum(q1, q2))
    u = jnp.concatenate(v, axis=-1)
    m = jnp.max(u, axis=-1, keepdims=True)
    for _ in range(3):
        m = jnp.max(jnp.where(u < m, u, _NEG), axis=-1, keepdims=True)
    return m


def _mask_kernel(q_ref, k_ref, out_ref, *, k_top):
    q = q_ref[0]
    k = k_ref[0]
    qn = q / jnp.maximum(jnp.sqrt(jnp.sum(q * q, axis=-1, keepdims=True)), 1e-12)
    kn = k / jnp.maximum(jnp.sqrt(jnp.sum(k * k, axis=-1, keepdims=True)), 1e-12)
    scores = jax.lax.dot_general(
        qn, kn, (((1,), (1,)), ((), ())),
        preferred_element_type=jnp.float32,
        precision=jax.lax.Precision.DEFAULT,
    )

    m = _rank4(scores)
    for _ in range(k_top // 4 - 1):
        m = _rank4(jnp.where(scores < m, scores, _NEG))
    out_ref[0] = (scores >= m).astype(jnp.float32)


def kernel(q_blocks, k_blocks):
    B, Qb, C = q_blocks.shape
    _, Bb, _ = k_blocks.shape
    k_top = min(_K_TOP, Bb)
    tq = min(512, Qb)
    grid = (B, Qb // tq)
    return pl.pallas_call(
        functools.partial(_mask_kernel, k_top=k_top),
        grid=grid,
        in_specs=[
            pl.BlockSpec((1, tq, C), lambda b, qt: (b, qt, 0)),
            pl.BlockSpec((1, Bb, C), lambda b, qt: (b, 0, 0)),
        ],
        out_specs=pl.BlockSpec((1, tq, Bb), lambda b, qt: (b, qt, 0)),
        out_shape=jax.ShapeDtypeStruct((B, Qb, Bb), jnp.float32),
        compiler_params=pltpu.CompilerParams(
            dimension_semantics=("parallel", "parallel"),
        ),
    )(q_blocks, k_blocks)


# speedup vs baseline: 1.2725x; 1.0603x over previous
import functools

import jax
import jax.numpy as jnp
from jax.experimental import pallas as pl
from jax.experimental.pallas import tpu as pltpu

_K_TOP = 16
_NEG = -3.0e38


def _rank4(x):
    h = x.shape[-1] // 2
    a, b = x[..., :h], x[..., h:]
    s1, s2 = jnp.maximum(a, b), jnp.minimum(a, b)
    h //= 2
    a1, b1 = s1[..., :h], s1[..., h:]
    a2, b2 = s2[..., :h], s2[..., h:]
    t = jnp.minimum(a1, b1)
    u = jnp.maximum(a2, b2)
    v = (jnp.maximum(a1, b1), jnp.maximum(t, u),
         jnp.minimum(t, u), jnp.minimum(a2, b2))
    while h > 128:
        h //= 2
        a = [c[..., :h] for c in v]
        b = [c[..., h:] for c in v]
        z1, z2 = jnp.maximum(a[0], b[3]), jnp.maximum(a[1], b[2])
        z3, z4 = jnp.maximum(a[2], b[1]), jnp.maximum(a[3], b[0])
        p1, p2 = jnp.maximum(z1, z3), jnp.maximum(z2, z4)
        q1, q2 = jnp.minimum(z1, z3), jnp.minimum(z2, z4)
        v = (jnp.maximum(p1, p2), jnp.minimum(p1, p2),
             jnp.maximum(q1, q2), jnp.minimum(q1, q2))
    o1, o2, o3, o4 = v
    m = jnp.max(o1, axis=-1, keepdims=True)
    for _ in range(3):
        adv = o1 == m
        o1 = jnp.where(adv, o2, o1)
        o2 = jnp.where(adv, o3, o2)
        o3 = jnp.where(adv, o4, o3)
        o4 = jnp.where(adv, _NEG, o4)
        m = jnp.max(o1, axis=-1, keepdims=True)
    return m


def _mask_kernel(q_ref, k_ref, out_ref, *, k_top):
    q = q_ref[0]
    k = k_ref[0]
    qn = q / jnp.maximum(jnp.sqrt(jnp.sum(q * q, axis=-1, keepdims=True)), 1e-12)
    kn = k / jnp.maximum(jnp.sqrt(jnp.sum(k * k, axis=-1, keepdims=True)), 1e-12)
    scores = jax.lax.dot_general(
        qn, kn, (((1,), (1,)), ((), ())),
        preferred_element_type=jnp.float32,
        precision=jax.lax.Precision.DEFAULT,
    )

    m = _rank4(scores)
    for _ in range(k_top // 4 - 1):
        m = _rank4(jnp.where(scores < m, scores, _NEG))
    out_ref[0] = (scores >= m).astype(jnp.float32)


def kernel(q_blocks, k_blocks):
    B, Qb, C = q_blocks.shape
    _, Bb, _ = k_blocks.shape
    k_top = min(_K_TOP, Bb)
    tq = min(512, Qb)
    grid = (B, Qb // tq)
    return pl.pallas_call(
        functools.partial(_mask_kernel, k_top=k_top),
        grid=grid,
        in_specs=[
            pl.BlockSpec((1, tq, C), lambda b, qt: (b, qt, 0)),
            pl.BlockSpec((1, Bb, C), lambda b, qt: (b, 0, 0)),
        ],
        out_specs=pl.BlockSpec((1, tq, Bb), lambda b, qt: (b, qt, 0)),
        out_shape=jax.ShapeDtypeStruct((B, Qb, Bb), jnp.float32),
        compiler_params=pltpu.CompilerParams(
            dimension_semantics=("parallel", "parallel"),
        ),
    )(q_blocks, k_blocks)


# speedup vs baseline: 1.2802x; 1.0060x over previous
import functools

import jax
import jax.numpy as jnp
from jax.experimental import pallas as pl
from jax.experimental.pallas import tpu as pltpu

_K_TOP = 16
_NEG = -3.0e38


def _ce(a, b):
    return jnp.maximum(a, b), jnp.minimum(a, b)


def _bitonic_sort(v):
    if len(v) == 1:
        return v
    half = len(v) // 2
    hi, lo = [], []
    for i in range(half):
        x, y = _ce(v[i], v[i + half])
        hi.append(x)
        lo.append(y)
    return _bitonic_sort(hi) + _bitonic_sort(lo)


def _merge(a, b):
    n = len(a)
    z, w = [], []
    for i in range(n):
        x, y = _ce(a[i], b[n - 1 - i])
        z.append(x)
        w.append(y)
    return _bitonic_sort(z) + _bitonic_sort(w)


def _rank_threshold(x, k_top):
    v = [x]
    while v[0].shape[-1] > 128 and len(v) < k_top:
        h = v[0].shape[-1] // 2
        v = _merge([c[..., :h] for c in v], [c[..., h:] for c in v])
    while v[0].shape[-1] > 128:
        h = v[0].shape[-1] // 2
        a = [c[..., :h] for c in v]
        b = [c[..., h:] for c in v]
        z = [jnp.maximum(a[i], b[k_top - 1 - i]) for i in range(k_top)]
        v = _bitonic_sort(z)
    neg = jnp.full_like(v[0], _NEG)
    v = v + [neg] * (k_top - len(v))
    m = jnp.max(v[0], axis=-1, keepdims=True)
    for k in range(k_top - 1):
        adv = v[0] == m
        for i in range(k_top - 1 - k):
            v[i] = jnp.where(adv, v[i + 1], v[i])
        m = jnp.max(v[0], axis=-1, keepdims=True)
    return m


def _mask_kernel(q_ref, k_ref, out_ref, *, k_top):
    q = q_ref[0]
    k = k_ref[0]
    qn = q / jnp.maximum(jnp.sqrt(jnp.sum(q * q, axis=-1, keepdims=True)), 1e-12)
    kn = k / jnp.maximum(jnp.sqrt(jnp.sum(k * k, axis=-1, keepdims=True)), 1e-12)
    scores = jax.lax.dot_general(
        qn, kn, (((1,), (1,)), ((), ())),
        preferred_element_type=jnp.float32,
        precision=jax.lax.Precision.DEFAULT,
    )
    thresh = _rank_threshold(scores, k_top)
    out_ref[0] = (scores >= thresh).astype(jnp.float32)


def kernel(q_blocks, k_blocks):
    B, Qb, C = q_blocks.shape
    _, Bb, _ = k_blocks.shape
    k_top = min(_K_TOP, Bb)
    tq = min(512, Qb)
    grid = (B, Qb // tq)
    return pl.pallas_call(
        functools.partial(_mask_kernel, k_top=k_top),
        grid=grid,
        in_specs=[
            pl.BlockSpec((1, tq, C), lambda b, qt: (b, qt, 0)),
            pl.BlockSpec((1, Bb, C), lambda b, qt: (b, 0, 0)),
        ],
        out_specs=pl.BlockSpec((1, tq, Bb), lambda b, qt: (b, qt, 0)),
        out_shape=jax.ShapeDtypeStruct((B, Qb, Bb), jnp.float32),
        compiler_params=pltpu.CompilerParams(
            dimension_semantics=("parallel", "parallel"),
        ),
    )(q_blocks, k_blocks)
